# Initial kernel scaffold; baseline (speedup 1.0000x reference)
import jax
import jax.numpy as jnp
from jax.experimental import pallas as pl


def kernel(x, edge_index, W1, b1, W2, b2, Wm, bm, Wl, bl):
    raise NotImplementedError("write your pallas kernel here")


# speedup vs baseline: 3.4762x; 3.4762x over previous
import functools

import jax
import jax.numpy as jnp
from jax import lax
from jax.experimental import pallas as pl
from jax.experimental.pallas import tpu as pltpu
from jax.experimental.pallas import tpu_sc as plsc

N = 10000
E = 160000
D_IN = 128
LATENT = 64
HID = 128

NC, NS = 2, 16
NW = NC * NS
NP = 10240
EP = 163840
EPT = EP // NW
CHUNK = 128
NCHUNK = EPT // CHUNK
ROWS_PT = NP // NS

_MESH = plsc.VectorSubcoreMesh(
    core_axis_name="c", subcore_axis_name="s", num_cores=NC, num_subcores=NS)


def _worker(cid, sid):
    return sid * NC + cid


def _hist_body(dst_hbm, ones_hbm, z_hbm, out_hbm, idx_v, ones_v, drain_v, hist_sh):
    cid = lax.axis_index("c")
    sid = lax.axis_index("s")
    wid = _worker(cid, sid)
    rows = pl.ds(sid * ROWS_PT, ROWS_PT)
    pltpu.sync_copy(z_hbm.at[rows], drain_v)
    pltpu.sync_copy(drain_v, hist_sh.at[rows])
    pltpu.sync_copy(ones_hbm, ones_v)
    plsc.subcore_barrier()

    def chunk(k, carry):
        base = wid * EPT + k * CHUNK
        pltpu.sync_copy(dst_hbm.at[pl.ds(base, CHUNK)], idx_v)
        pltpu.sync_copy(ones_v, hist_sh.at[idx_v], add=True)
        return carry

    if False:
        lax.fori_loop(0, NCHUNK, chunk, 0)
    plsc.subcore_barrier()
    pltpu.sync_copy(hist_sh.at[rows], drain_v)
    pltpu.sync_copy(drain_v, out_hbm.at[cid, rows])


_hist_call = pl.kernel(
    _hist_body,
    out_type=jax.ShapeDtypeStruct((NC, NP, 16), jnp.float32),
    mesh=_MESH,
    scratch_types=[
        pltpu.VMEM((CHUNK,), jnp.int32),
        pltpu.VMEM((CHUNK, 16), jnp.float32),
        pltpu.VMEM((ROWS_PT, 16), jnp.float32),
        pltpu.VMEM_SHARED((NP, 16), jnp.float32),
    ],
)


def _agg_body(y_hbm, src_hbm, dst_hbm, z_hbm, out_hbm,
              sidx_v, didx_v, rows_v, acc_sh, sem, *, depth):
    cid = lax.axis_index("c")
    sid = lax.axis_index("s")
    wid = _worker(cid, sid)
    for j in range(ROWS_PT // CHUNK):
        rows = pl.ds(sid * ROWS_PT + j * CHUNK, CHUNK)
        pltpu.sync_copy(z_hbm.at[rows], rows_v)
        pltpu.sync_copy(rows_v, acc_sh.at[rows])
    plsc.subcore_barrier()

    def chunk(k, carry):
        base = wid * EPT + k * CHUNK
        pltpu.sync_copy(src_hbm.at[pl.ds(base, CHUNK)], sidx_v)
        pltpu.sync_copy(dst_hbm.at[pl.ds(base, CHUNK)], didx_v)
        pltpu.async_copy(y_hbm.at[sidx_v], rows_v, sem).wait()
        pltpu.sync_copy(rows_v, acc_sh.at[didx_v], add=True)
        return carry

    lax.fori_loop(0, NCHUNK, chunk, 0)
    plsc.subcore_barrier()
    for j in range(ROWS_PT // CHUNK):
        rows = pl.ds(sid * ROWS_PT + j * CHUNK, CHUNK)
        pltpu.sync_copy(acc_sh.at[rows], rows_v)
        pltpu.sync_copy(rows_v, out_hbm.at[cid, rows])


def _make_agg(depth):
    return pl.kernel(
        functools.partial(_agg_body, depth=depth),
        out_type=jax.ShapeDtypeStruct((NC, NP, depth), jnp.float32),
        mesh=_MESH,
        scratch_types=[
            pltpu.VMEM((CHUNK,), jnp.int32),
            pltpu.VMEM((CHUNK,), jnp.int32),
            pltpu.VMEM((CHUNK, depth), jnp.float32),
            pltpu.VMEM_SHARED((NP, depth), jnp.float32),
            pltpu.SemaphoreType.DMA,
        ],
    )


_agg128 = _make_agg(HID)


_BR = 256
_GRID = NP // _BR


def _t1_body(hist_ref, x_ref, w1_ref, y1_ref, dinv_ref):
    deg = hist_ref[0, :, 0:1] + hist_ref[1, :, 0:1] + 2.0
    dinv = lax.rsqrt(deg)
    xw = jnp.dot(x_ref[...], w1_ref[...], preferred_element_type=jnp.float32)
    y1_ref[...] = xw * dinv
    dinv_ref[...] = dinv


def _t2_body(acc_ref, y1_ref, dinv_ref, w2_ref, b1_ref, y2_ref):
    dinv = dinv_ref[...]
    h = dinv * (acc_ref[0] + acc_ref[1] + 2.0 * y1_ref[...]) + b1_ref[...]
    h = jnp.maximum(h, 0.0)
    y2_ref[...] = dinv * jnp.dot(h, w2_ref[...], preferred_element_type=jnp.float32)


def _t3_body(acc_ref, y2_ref, dinv_ref, wm_ref, wl_ref, b2_ref, bm_ref, bl_ref,
             mu_ref, lv_ref):
    h2 = dinv_ref[...] * (acc_ref[0] + acc_ref[1] + 2.0 * y2_ref[...]) + b2_ref[...]
    mu_ref[...] = jnp.dot(h2, wm_ref[...], preferred_element_type=jnp.float32) + bm_ref[...]
    lv_ref[...] = jnp.dot(h2, wl_ref[...], preferred_element_type=jnp.float32) + bl_ref[...]


_DB = 512


def _dec_body(mu_i_ref, mu_j_ref, adj_ref):
    prod = lax.dot_general(mu_i_ref[...], mu_j_ref[...],
                           (((1,), (1,)), ((), ())),
                           preferred_element_type=jnp.float32)
    adj_ref[...] = jax.nn.sigmoid(prod)


def _probe_body(table_hbm, idx_hbm, out_hbm, idx_v, rows_v, shared_sh, sem):
    cid = lax.axis_index("c")
    sid = lax.axis_index("s")
    wid = _worker(cid, sid)
    base = wid * 8
    srows = pl.ds(sid * 8, 8)
    pltpu.sync_copy(idx_hbm.at[pl.ds(base, 8)], idx_v)
    pltpu.async_copy(table_hbm.at[idx_v], rows_v, sem).wait()
    pltpu.sync_copy(rows_v, shared_sh.at[srows])
    plsc.subcore_barrier()
    pltpu.sync_copy(shared_sh.at[srows], rows_v)
    pltpu.sync_copy(rows_v, out_hbm.at[cid, srows])


_probe_call = pl.kernel(
    _probe_body,
    out_type=jax.ShapeDtypeStruct((NC, NS * 8, HID), jnp.float32),
    mesh=_MESH,
    scratch_types=[
        pltpu.VMEM((8,), jnp.int32),
        pltpu.VMEM((8, HID), jnp.float32),
        pltpu.VMEM_SHARED((NS * 8, HID), jnp.float32),
        pltpu.SemaphoreType.DMA,
    ],
)


def kernel(x, edge_index, W1, b1, W2, b2, Wm, bm, Wl, bl):
    src = edge_index[0]
    dst = edge_index[1]
    pad_e = EP - E
    srcp = jnp.concatenate([src, jnp.zeros((pad_e,), jnp.int32)])
    dstp = jnp.concatenate([dst, jnp.full((pad_e,), N, jnp.int32)])
    xp = jnp.pad(x, ((0, NP - N), (0, 0)))
    z128 = jnp.zeros((NP, HID), jnp.float32)
    W2p = jnp.pad(W2, ((0, 0), (0, HID - LATENT)))

    ones_tbl = jnp.zeros((NP, HID), jnp.float32).at[:, 0].set(1.0)
    hist = _agg128(ones_tbl, srcp, dstp, z128)

    y1, dinv = pl.pallas_call(
        _t1_body,
        grid=(_GRID,),
        in_specs=[
            pl.BlockSpec((NC, _BR, HID), lambda i: (0, i, 0)),
            pl.BlockSpec((_BR, D_IN), lambda i: (i, 0)),
            pl.BlockSpec((D_IN, HID), lambda i: (0, 0)),
        ],
        out_specs=[
            pl.BlockSpec((_BR, HID), lambda i: (i, 0)),
            pl.BlockSpec((_BR, 1), lambda i: (i, 0)),
        ],
        out_shape=[
            jax.ShapeDtypeStruct((NP, HID), jnp.float32),
            jax.ShapeDtypeStruct((NP, 1), jnp.float32),
        ],
    )(hist, xp, W1)

    acc1 = _agg128(y1, srcp, dstp, z128)

    y2 = pl.pallas_call(
        _t2_body,
        grid=(_GRID,),
        in_specs=[
            pl.BlockSpec((NC, _BR, HID), lambda i: (0, i, 0)),
            pl.BlockSpec((_BR, HID), lambda i: (i, 0)),
            pl.BlockSpec((_BR, 1), lambda i: (i, 0)),
            pl.BlockSpec((HID, HID), lambda i: (0, 0)),
            pl.BlockSpec((1, HID), lambda i: (0, 0)),
        ],
        out_specs=pl.BlockSpec((_BR, HID), lambda i: (i, 0)),
        out_shape=jax.ShapeDtypeStruct((NP, HID), jnp.float32),
    )(acc1, y1, dinv, W2p, b1.reshape(1, HID))

    acc2 = _agg128(y2, srcp, dstp, z128)

    mu_p, lv_p = pl.pallas_call(
        _t3_body,
        grid=(_GRID,),
        in_specs=[
            pl.BlockSpec((NC, _BR, HID), lambda i: (0, i, 0)),
            pl.BlockSpec((_BR, HID), lambda i: (i, 0)),
            pl.BlockSpec((_BR, 1), lambda i: (i, 0)),
            pl.BlockSpec((HID, LATENT), lambda i: (0, 0)),
            pl.BlockSpec((HID, LATENT), lambda i: (0, 0)),
            pl.BlockSpec((1, HID), lambda i: (0, 0)),
            pl.BlockSpec((1, LATENT), lambda i: (0, 0)),
            pl.BlockSpec((1, LATENT), lambda i: (0, 0)),
        ],
        out_specs=[
            pl.BlockSpec((_BR, LATENT), lambda i: (i, 0)),
            pl.BlockSpec((_BR, LATENT), lambda i: (i, 0)),
        ],
        out_shape=[
            jax.ShapeDtypeStruct((NP, LATENT), jnp.float32),
            jax.ShapeDtypeStruct((NP, LATENT), jnp.float32),
        ],
    )(acc2, y2, dinv,
      jnp.pad(Wm, ((0, HID - LATENT), (0, 0))),
      jnp.pad(Wl, ((0, HID - LATENT), (0, 0))),
      jnp.pad(b2, (0, HID - LATENT)).reshape(1, HID),
      bm.reshape(1, LATENT), bl.reshape(1, LATENT))

    adj = pl.pallas_call(
        _dec_body,
        grid=(N // _DB + 1, N // _DB + 1),
        in_specs=[
            pl.BlockSpec((_DB, LATENT), lambda i, j: (i, 0)),
            pl.BlockSpec((_DB, LATENT), lambda i, j: (j, 0)),
        ],
        out_specs=pl.BlockSpec((_DB, _DB), lambda i, j: (i, j)),
        out_shape=jax.ShapeDtypeStruct((N, N), jnp.float32),
    )(mu_p, mu_p)

    return (adj, mu_p[:N], lv_p[:N])
